# baseline (speedup 1.0000x reference)
import jax
import jax.numpy as jnp
from jax.experimental import pallas as pl
from jax.experimental.pallas import tpu as pltpu

_KMAX = 50
_KMIN = 50
_SIGN_MASK = 0x7FFFFFFF
_INT_MIN = -2147483648
_INT_MAX = 2147483647


def _select_body(x_ref, o_ref, keys_ref):
    rows, n = x_ref.shape
    bits = jax.lax.bitcast_convert_type(x_ref[...], jnp.int32)
    keys_ref[...] = jnp.where(bits >= 0, bits, bits ^ _SIGN_MASK)

    nsplit = 8
    cseg = n // nsplit

    def counts(t, u):
        cts, cus = [], []
        for s in range(nsplit):
            k = keys_ref[:, s * cseg:(s + 1) * cseg]
            cts.append(jnp.sum((k >= t).astype(jnp.int32), axis=1,
                               keepdims=True))
            cus.append(jnp.sum((k <= u).astype(jnp.int32), axis=1,
                               keepdims=True))
        return sum(cts), sum(cus)

    def search_body(nbits):
        top = 1 << (nbits - 1)

        def body(i, carry):
            t, u = carry
            p = (top >> i).astype(jnp.int32)
            tt = t + p
            uu = u - p
            ct, cu = counts(tt, uu)
            return (jnp.where(ct >= _KMAX, tt, t),
                    jnp.where(cu >= _KMIN, uu, u))

        return body

    maxs, mins = [], []
    for s in range(nsplit):
        k = keys_ref[:, s * cseg:(s + 1) * cseg]
        maxs.append(jnp.max(k, axis=1, keepdims=True))
        mins.append(jnp.min(k, axis=1, keepdims=True))
    while len(maxs) > 1:
        maxs = [jnp.maximum(a, b) for a, b in zip(maxs[::2], maxs[1::2])]
        mins = [jnp.minimum(a, b) for a, b in zip(mins[::2], mins[1::2])]
    maxk = maxs[0]
    mink = mins[0]
    win = (1 << 23) - 1
    t0h = maxk - win
    u0h = mink + win
    ct0, cu0 = counts(t0h, u0h)
    hint_ok = jnp.logical_and(jnp.all(ct0 >= _KMAX), jnp.all(cu0 >= _KMIN))

    def short_search(_):
        return jax.lax.fori_loop(0, 23, search_body(23), (t0h, u0h))

    def full_search(_):
        zero = jnp.zeros((rows, 1), jnp.int32)
        ct, cu = counts(zero, zero - 1)
        t0 = jnp.where(ct >= _KMAX, zero, zero + _INT_MIN)
        u0 = jnp.where(cu >= _KMIN, zero - 1, zero + _INT_MAX)
        return jax.lax.fori_loop(0, 31, search_body(31), (t0, u0))

    t, u = jax.lax.cond(hint_ok, short_search, full_search, None)

    cnt_gt, cnt_lt = counts(t + 1, u - 1)
    seg = n // 8
    gs, ls = [], []
    for sidx in range(8):
        ks = keys_ref[:, sidx * seg:(sidx + 1) * seg]
        xs = x_ref[:, sidx * seg:(sidx + 1) * seg]
        gs.append(jnp.sum(jnp.where(ks > t, xs, 0.0), axis=1, keepdims=True))
        ls.append(jnp.sum(jnp.where(ks < u, xs, 0.0), axis=1, keepdims=True))
    s_gt = sum(gs)
    s_lt = sum(ls)
    tval = jax.lax.bitcast_convert_type(
        jnp.where(t >= 0, t, t ^ _SIGN_MASK), jnp.float32)
    uval = jax.lax.bitcast_convert_type(
        jnp.where(u >= 0, u, u ^ _SIGN_MASK), jnp.float32)
    top = s_gt + (_KMAX - cnt_gt).astype(jnp.float32) * tval
    bot = s_lt + (_KMIN - cnt_lt).astype(jnp.float32) * uval
    res = top / _KMAX + bot / _KMIN
    o_ref[0] = jnp.broadcast_to(res, (rows, 128))


from jax import lax
from jax.experimental.pallas import tpu_sc as plsc
import functools

_K = 50
_WIN = (1 << 23) - 1
_CAP = 16384
_CBUF = _CAP + 80
_NW = 32


def _splat_last(v):
    idx = jnp.full((16, 1), 15, jnp.int32)
    dn = lax.GatherDimensionNumbers(offset_dims=(), collapsed_slice_dims=(0,),
                                    start_index_map=(0,))
    return lax.gather(v, idx, dn, (1,),
                      mode=lax.GatherScatterMode.PROMISE_IN_BOUNDS)


def _splat_max(v):
    return _splat_last(plsc.cummax(v))


def _splat_min(v):
    return -_splat_last(plsc.cummax(-v))


def _splat_sum(v):
    return _splat_last(plsc.cumsum(v))


def _scal(v):
    return v[0]


def _f_to_key(v):
    b = plsc.bitcast(v, jnp.int32)
    return jnp.where(b >= 0, b, b ^ 0x7FFFFFFF)


def _key_to_f(t):
    return plsc.bitcast(jnp.where(t >= 0, t, t ^ 0x7FFFFFFF), jnp.float32)


def _count_ge(vl, nout, unroll, thf):
    zero = jnp.zeros((16,), jnp.int32)

    def body(j, acc):
        for k in range(unroll):
            acc = acc + plsc.all_reduce_population_count(
                vl(j * unroll + k) >= thf)
        return acc

    return lax.fori_loop(0, nout, body, zero)


def _select_top(vl, nout, unroll, t0, nbits):
    def sb(i, t):
        p = jnp.left_shift(jnp.int32(1), (nbits - 1) - i)
        tt = t + p
        c = _count_ge(vl, nout, unroll, _key_to_f(tt))
        ok = jnp.logical_and(tt > t, c >= _K)
        return jnp.where(ok, tt, t)

    return lax.fori_loop(0, nbits, sb, t0)


def _top_sum(vl, nout, unroll, tkey):
    thf = _key_to_f(tkey)
    zf = jnp.zeros((16,), jnp.float32)
    zi = jnp.zeros((16,), jnp.int32)

    def body(j, carry):
        s, c = carry
        for k in range(unroll):
            v = vl(j * unroll + k)
            m = v > thf
            s = s + jnp.where(m, v, 0.0)
            c = c + plsc.all_reduce_population_count(m)
        return s, c

    s, c = lax.fori_loop(0, nout, body, (zf, zi))
    return _splat_sum(s) + (_K - c).astype(jnp.float32) * thf


def _make_sc_kernel(rows, n):
    rpw = rows // _NW
    nvec = n // 16
    nout8 = nvec // 8
    mesh = plsc.VectorSubcoreMesh(core_axis_name="c", subcore_axis_name="s")

    @functools.partial(
        pl.kernel, mesh=mesh,
        compiler_params=pltpu.CompilerParams(needs_layout_passes=False),
        out_type=jax.ShapeDtypeStruct((rows * 16,), jnp.float32),
        scratch_types=[
            pltpu.VMEM((n,), jnp.float32),
            pltpu.VMEM((_CBUF,), jnp.float32),
            pltpu.VMEM((_CBUF,), jnp.float32),
            pltpu.VMEM((rpw * 16,), jnp.float32),
        ],
    )
    def sc_kernel(x_hbm, out_hbm, row_v, cand_t, cand_b, out_loc):
        wid = lax.axis_index("s") * 2 + lax.axis_index("c")
        base = wid * rpw
        iota16 = lax.iota(jnp.int32, 16)
        ninf = jnp.full((16,), -jnp.inf, jnp.float32)

        def vl_row(j):
            return row_v[pl.ds(j * 16, 16)]

        def vl_rown(j):
            return -row_v[pl.ds(j * 16, 16)]

        def vl_ct(j):
            return cand_t[pl.ds(j * 16, 16)]

        def vl_cb(j):
            return cand_b[pl.ds(j * 16, 16)]

        def row_loop(r, carry):
            pltpu.sync_copy(x_hbm.at[base + r], row_v)

            def mm(j, mc):
                mx, mn = mc
                for k in range(8):
                    v = vl_row(j * 8 + k)
                    mx = jnp.maximum(mx, v)
                    mn = jnp.minimum(mn, v)
                return mx, mn

            mx, mn = lax.fori_loop(0, nout8, mm, (ninf, -ninf))
            mxv = _splat_max(mx)
            mnv = _splat_min(mn)
            t0t = _f_to_key(mxv) - _WIN
            t0b = _f_to_key(-mnv) - _WIN
            tht = _key_to_f(t0t)
            thb = _key_to_f(t0b)

            def ap(j, bc):
                bt, bb = bc
                for k in range(8):
                    v = vl_row(j * 8 + k)
                    nv = -v
                    mt = v >= tht
                    mb = nv >= thb
                    plsc.store_compressed(
                        cand_t.at[pl.ds(jnp.minimum(bt, _CAP), 16)], v, mask=mt)
                    bt = bt + _scal(plsc.all_reduce_population_count(mt))
                    plsc.store_compressed(
                        cand_b.at[pl.ds(jnp.minimum(bb, _CAP), 16)], nv, mask=mb)
                    bb = bb + _scal(plsc.all_reduce_population_count(mb))
                return bt, bb

            zs = jnp.zeros((), jnp.int32)
            bt, bb = lax.fori_loop(0, nout8, ap, (zs, zs))
            for k in range(4):
                cand_t[pl.ds(jnp.minimum(bt + 16 * k, _CBUF - 16), 16)] = ninf
                cand_b[pl.ds(jnp.minimum(bb + 16 * k, _CBUF - 16), 16)] = ninf
            nct = bt
            ncb = bb

            def cand_sum(vl, nc, t0):
                nout4 = (nc + 63) >> 6
                tk = _select_top(vl, nout4, 4, t0, 23)
                return _top_sum(vl, nout4, 4, tk)

            def fb_sum(vl, minv):
                t0 = _f_to_key(minv)
                tk = _select_top(vl, nout8, 8, t0, 32)
                return _top_sum(vl, nout8, 8, tk)

            ts = lax.cond(
                jnp.logical_and(nct >= _K, nct <= _CAP),
                lambda _: cand_sum(vl_ct, nct, t0t),
                lambda _: fb_sum(vl_row, mnv), None)
            bs = lax.cond(
                jnp.logical_and(ncb >= _K, ncb <= _CAP),
                lambda _: cand_sum(vl_cb, ncb, t0b),
                lambda _: fb_sum(vl_rown, -mxv), None)

            res = ts / _K - bs / _K
            out_loc[pl.ds(r * 16, 16)] = res
            return carry

        lax.fori_loop(0, rpw, row_loop, 0)
        pltpu.sync_copy(out_loc, out_hbm.at[pl.ds(base * 16, rpw * 16)])

    return sc_kernel


def _tc_call(x, rows, n):
    rg = 16
    g = rows // rg
    out = pl.pallas_call(
        _select_body,
        grid=(g,),
        in_specs=[pl.BlockSpec((rg, n), lambda i: (i, 0))],
        out_specs=pl.BlockSpec((1, rg, 128), lambda i: (i, 0, 0)),
        out_shape=jax.ShapeDtypeStruct((g, rg, 128), jnp.float32),
        scratch_shapes=[pltpu.VMEM((rg, n), jnp.int32)],
    )(x)
    return out[:, :, 0].reshape(rows)


def kernel(inputs):
    b, h, w, c = inputs.shape
    n = h * w
    rows = b * c
    x = jnp.transpose(inputs, (0, 3, 1, 2)).reshape(rows, n)
    rt = (rows * 50 // 100) // 16 * 16
    rs = rows - rt
    if rs % _NW != 0:
        rs -= rs % _NW
        rt = rows - rs
    out_sc = _make_sc_kernel(rs, n)(x[rt:])
    out_tc = _tc_call(x[:rt], rt, n)
    out = jnp.concatenate([out_tc, out_sc.reshape(rs, 16)[:, 0]], axis=0)
    return out.reshape(b, c)


# speedup vs baseline: 1.0458x; 1.0458x over previous
import jax
import jax.numpy as jnp
from jax.experimental import pallas as pl
from jax.experimental.pallas import tpu as pltpu

_KMAX = 50
_KMIN = 50
_SIGN_MASK = 0x7FFFFFFF
_INT_MIN = -2147483648
_INT_MAX = 2147483647


def _select_body(x_ref, o_ref, keys_ref):
    rows, n = x_ref.shape
    bits = jax.lax.bitcast_convert_type(x_ref[...], jnp.int32)
    keys_ref[...] = jnp.where(bits >= 0, bits, bits ^ _SIGN_MASK)

    nsplit = 8
    cseg = n // nsplit

    def counts(t, u):
        cts, cus = [], []
        for s in range(nsplit):
            k = keys_ref[:, s * cseg:(s + 1) * cseg]
            cts.append(jnp.sum((k >= t).astype(jnp.int32), axis=1,
                               keepdims=True))
            cus.append(jnp.sum((k <= u).astype(jnp.int32), axis=1,
                               keepdims=True))
        return sum(cts), sum(cus)

    def search_body(nbits):
        top = 1 << (nbits - 1)

        def body(i, carry):
            t, u = carry
            p = (top >> i).astype(jnp.int32)
            tt = t + p
            uu = u - p
            ct, cu = counts(tt, uu)
            return (jnp.where(ct >= _KMAX, tt, t),
                    jnp.where(cu >= _KMIN, uu, u))

        return body

    maxs, mins = [], []
    for s in range(nsplit):
        k = keys_ref[:, s * cseg:(s + 1) * cseg]
        maxs.append(jnp.max(k, axis=1, keepdims=True))
        mins.append(jnp.min(k, axis=1, keepdims=True))
    while len(maxs) > 1:
        maxs = [jnp.maximum(a, b) for a, b in zip(maxs[::2], maxs[1::2])]
        mins = [jnp.minimum(a, b) for a, b in zip(mins[::2], mins[1::2])]
    maxk = maxs[0]
    mink = mins[0]
    win = (1 << 23) - 1
    t0h = maxk - win
    u0h = mink + win
    ct0, cu0 = counts(t0h, u0h)
    hint_ok = jnp.logical_and(jnp.all(ct0 >= _KMAX), jnp.all(cu0 >= _KMIN))

    def short_search(_):
        return jax.lax.fori_loop(0, 23, search_body(23), (t0h, u0h))

    def full_search(_):
        zero = jnp.zeros((rows, 1), jnp.int32)
        ct, cu = counts(zero, zero - 1)
        t0 = jnp.where(ct >= _KMAX, zero, zero + _INT_MIN)
        u0 = jnp.where(cu >= _KMIN, zero - 1, zero + _INT_MAX)
        return jax.lax.fori_loop(0, 31, search_body(31), (t0, u0))

    t, u = jax.lax.cond(hint_ok, short_search, full_search, None)

    cnt_gt, cnt_lt = counts(t + 1, u - 1)
    seg = n // 8
    gs, ls = [], []
    for sidx in range(8):
        ks = keys_ref[:, sidx * seg:(sidx + 1) * seg]
        xs = x_ref[:, sidx * seg:(sidx + 1) * seg]
        gs.append(jnp.sum(jnp.where(ks > t, xs, 0.0), axis=1, keepdims=True))
        ls.append(jnp.sum(jnp.where(ks < u, xs, 0.0), axis=1, keepdims=True))
    s_gt = sum(gs)
    s_lt = sum(ls)
    tval = jax.lax.bitcast_convert_type(
        jnp.where(t >= 0, t, t ^ _SIGN_MASK), jnp.float32)
    uval = jax.lax.bitcast_convert_type(
        jnp.where(u >= 0, u, u ^ _SIGN_MASK), jnp.float32)
    top = s_gt + (_KMAX - cnt_gt).astype(jnp.float32) * tval
    bot = s_lt + (_KMIN - cnt_lt).astype(jnp.float32) * uval
    res = top / _KMAX + bot / _KMIN
    o_ref[0] = jnp.broadcast_to(res, (rows, 128))


from jax import lax
from jax.experimental.pallas import tpu_sc as plsc
import functools

_K = 50
_WIN = (1 << 23) - 1
_CAP = 16384
_CBUF = _CAP + 80
_NW = 32


def _splat_last(v):
    idx = jnp.full((16, 1), 15, jnp.int32)
    dn = lax.GatherDimensionNumbers(offset_dims=(), collapsed_slice_dims=(0,),
                                    start_index_map=(0,))
    return lax.gather(v, idx, dn, (1,),
                      mode=lax.GatherScatterMode.PROMISE_IN_BOUNDS)


def _splat_max(v):
    return _splat_last(plsc.cummax(v))


def _splat_min(v):
    return -_splat_last(plsc.cummax(-v))


def _splat_sum(v):
    return _splat_last(plsc.cumsum(v))


def _scal(v):
    return v[0]


def _f_to_key(v):
    b = plsc.bitcast(v, jnp.int32)
    return jnp.where(b >= 0, b, b ^ 0x7FFFFFFF)


def _key_to_f(t):
    return plsc.bitcast(jnp.where(t >= 0, t, t ^ 0x7FFFFFFF), jnp.float32)


def _count_ge(vl, nout, unroll, thf):
    zero = jnp.zeros((16,), jnp.int32)

    def body(j, acc):
        for k in range(unroll):
            acc = acc + plsc.all_reduce_population_count(
                vl(j * unroll + k) >= thf)
        return acc

    return lax.fori_loop(0, nout, body, zero)


def _select_top(vl, nout, unroll, t0, nbits):
    def sb(i, t):
        p = jnp.left_shift(jnp.int32(1), (nbits - 1) - i)
        tt = t + p
        c = _count_ge(vl, nout, unroll, _key_to_f(tt))
        ok = jnp.logical_and(tt > t, c >= _K)
        return jnp.where(ok, tt, t)

    return lax.fori_loop(0, nbits, sb, t0)


def _top_sum(vl, nout, unroll, tkey):
    thf = _key_to_f(tkey)
    zf = jnp.zeros((16,), jnp.float32)
    zi = jnp.zeros((16,), jnp.int32)

    def body(j, carry):
        s, c = carry
        for k in range(unroll):
            v = vl(j * unroll + k)
            m = v > thf
            s = s + jnp.where(m, v, 0.0)
            c = c + plsc.all_reduce_population_count(m)
        return s, c

    s, c = lax.fori_loop(0, nout, body, (zf, zi))
    return _splat_sum(s) + (_K - c).astype(jnp.float32) * thf


def _make_sc_kernel(rows, n):
    rpw = rows // _NW
    nvec = n // 16
    nout8 = nvec // 8
    mesh = plsc.VectorSubcoreMesh(core_axis_name="c", subcore_axis_name="s")

    @functools.partial(
        pl.kernel, mesh=mesh,
        compiler_params=pltpu.CompilerParams(needs_layout_passes=False),
        out_type=jax.ShapeDtypeStruct((rows * 16,), jnp.float32),
        scratch_types=[
            pltpu.VMEM((n,), jnp.float32),
            pltpu.VMEM((_CBUF,), jnp.float32),
            pltpu.VMEM((_CBUF,), jnp.float32),
            pltpu.VMEM((rpw * 16,), jnp.float32),
        ],
    )
    def sc_kernel(x_hbm, out_hbm, row_v, cand_t, cand_b, out_loc):
        wid = lax.axis_index("s") * 2 + lax.axis_index("c")
        base = wid * rpw
        iota16 = lax.iota(jnp.int32, 16)
        ninf = jnp.full((16,), -jnp.inf, jnp.float32)

        def vl_row(j):
            return row_v[pl.ds(j * 16, 16)]

        def vl_rown(j):
            return -row_v[pl.ds(j * 16, 16)]

        def vl_ct(j):
            return cand_t[pl.ds(j * 16, 16)]

        def vl_cb(j):
            return cand_b[pl.ds(j * 16, 16)]

        def row_loop(r, carry):
            pltpu.sync_copy(x_hbm.at[base + r], row_v)

            def mm(j, mc):
                mx, mn = mc
                for k in range(8):
                    v = vl_row(j * 8 + k)
                    mx = jnp.maximum(mx, v)
                    mn = jnp.minimum(mn, v)
                return mx, mn

            mx, mn = lax.fori_loop(0, nout8, mm, (ninf, -ninf))
            mxv = _splat_max(mx)
            mnv = _splat_min(mn)
            t0t = _f_to_key(mxv) - _WIN
            t0b = _f_to_key(-mnv) - _WIN
            tht = _key_to_f(t0t)
            thb = _key_to_f(t0b)

            def ap(j, bc):
                bt, bb = bc
                for k in range(8):
                    v = vl_row(j * 8 + k)
                    nv = -v
                    mt = v >= tht
                    mb = nv >= thb
                    plsc.store_compressed(
                        cand_t.at[pl.ds(jnp.minimum(bt, _CAP), 16)], v, mask=mt)
                    bt = bt + _scal(plsc.all_reduce_population_count(mt))
                    plsc.store_compressed(
                        cand_b.at[pl.ds(jnp.minimum(bb, _CAP), 16)], nv, mask=mb)
                    bb = bb + _scal(plsc.all_reduce_population_count(mb))
                return bt, bb

            zs = jnp.zeros((), jnp.int32)
            bt, bb = lax.fori_loop(0, nout8, ap, (zs, zs))
            for k in range(4):
                cand_t[pl.ds(jnp.minimum(bt + 16 * k, _CBUF - 16), 16)] = ninf
                cand_b[pl.ds(jnp.minimum(bb + 16 * k, _CBUF - 16), 16)] = ninf
            nct = bt
            ncb = bb

            def cand_sum(vl, nc, t0):
                nout4 = (nc + 63) >> 6
                tk = _select_top(vl, nout4, 4, t0, 23)
                return _top_sum(vl, nout4, 4, tk)

            def fb_sum(vl, minv):
                t0 = _f_to_key(minv)
                tk = _select_top(vl, nout8, 8, t0, 32)
                return _top_sum(vl, nout8, 8, tk)

            ts = lax.cond(
                jnp.logical_and(nct >= _K, nct <= _CAP),
                lambda _: cand_sum(vl_ct, nct, t0t),
                lambda _: fb_sum(vl_row, mnv), None)
            bs = lax.cond(
                jnp.logical_and(ncb >= _K, ncb <= _CAP),
                lambda _: cand_sum(vl_cb, ncb, t0b),
                lambda _: fb_sum(vl_rown, -mxv), None)

            res = ts / _K - bs / _K
            out_loc[pl.ds(r * 16, 16)] = res
            return carry

        lax.fori_loop(0, rpw, row_loop, 0)
        pltpu.sync_copy(out_loc, out_hbm.at[pl.ds(base * 16, rpw * 16)])

    return sc_kernel


def _tc_call(x, rows, n):
    rg = 16
    g = rows // rg
    out = pl.pallas_call(
        _select_body,
        grid=(g,),
        in_specs=[pl.BlockSpec((rg, n), lambda i: (i, 0))],
        out_specs=pl.BlockSpec((1, rg, 128), lambda i: (i, 0, 0)),
        out_shape=jax.ShapeDtypeStruct((g, rg, 128), jnp.float32),
        scratch_shapes=[pltpu.VMEM((rg, n), jnp.int32)],
    )(x)
    return out[:, :, 0].reshape(rows)


def kernel(inputs):
    b, h, w, c = inputs.shape
    n = h * w
    rows = b * c
    x = jnp.transpose(inputs, (0, 3, 1, 2)).reshape(rows, n)
    rt = (rows * 58 // 100) // 16 * 16
    rs = rows - rt
    if rs % _NW != 0:
        rs -= rs % _NW
        rt = rows - rs
    out_sc = _make_sc_kernel(rs, n)(x[rt:])
    out_tc = _tc_call(x[:rt], rt, n)
    out = jnp.concatenate([out_tc, out_sc.reshape(rs, 16)[:, 0]], axis=0)
    return out.reshape(b, c)


# speedup vs baseline: 1.0993x; 1.0512x over previous
import jax
import jax.numpy as jnp
from jax.experimental import pallas as pl
from jax.experimental.pallas import tpu as pltpu

_KMAX = 50
_KMIN = 50
_SIGN_MASK = 0x7FFFFFFF
_INT_MIN = -2147483648
_INT_MAX = 2147483647


def _select_body(x_ref, o_ref, keys_ref):
    rows, n = x_ref.shape
    bits = jax.lax.bitcast_convert_type(x_ref[...], jnp.int32)
    keys_ref[...] = jnp.where(bits >= 0, bits, bits ^ _SIGN_MASK)

    nsplit = 8
    cseg = n // nsplit

    def counts(t, u):
        cts, cus = [], []
        for s in range(nsplit):
            k = keys_ref[:, s * cseg:(s + 1) * cseg]
            cts.append(jnp.sum((k >= t).astype(jnp.int32), axis=1,
                               keepdims=True))
            cus.append(jnp.sum((k <= u).astype(jnp.int32), axis=1,
                               keepdims=True))
        return sum(cts), sum(cus)

    def search_body(nbits):
        top = 1 << (nbits - 1)

        def body(i, carry):
            t, u = carry
            p = (top >> i).astype(jnp.int32)
            tt = t + p
            uu = u - p
            ct, cu = counts(tt, uu)
            return (jnp.where(ct >= _KMAX, tt, t),
                    jnp.where(cu >= _KMIN, uu, u))

        return body

    maxs, mins = [], []
    for s in range(nsplit):
        k = keys_ref[:, s * cseg:(s + 1) * cseg]
        maxs.append(jnp.max(k, axis=1, keepdims=True))
        mins.append(jnp.min(k, axis=1, keepdims=True))
    while len(maxs) > 1:
        maxs = [jnp.maximum(a, b) for a, b in zip(maxs[::2], maxs[1::2])]
        mins = [jnp.minimum(a, b) for a, b in zip(mins[::2], mins[1::2])]
    maxk = maxs[0]
    mink = mins[0]
    win = (1 << 23) - 1
    t0h = maxk - win
    u0h = mink + win
    ct0, cu0 = counts(t0h, u0h)
    hint_ok = jnp.logical_and(jnp.all(ct0 >= _KMAX), jnp.all(cu0 >= _KMIN))

    def short_search(_):
        return jax.lax.fori_loop(0, 23, search_body(23), (t0h, u0h))

    def full_search(_):
        zero = jnp.zeros((rows, 1), jnp.int32)
        ct, cu = counts(zero, zero - 1)
        t0 = jnp.where(ct >= _KMAX, zero, zero + _INT_MIN)
        u0 = jnp.where(cu >= _KMIN, zero - 1, zero + _INT_MAX)
        return jax.lax.fori_loop(0, 31, search_body(31), (t0, u0))

    t, u = jax.lax.cond(hint_ok, short_search, full_search, None)

    cnt_gt, cnt_lt = counts(t + 1, u - 1)
    seg = n // 8
    gs, ls = [], []
    for sidx in range(8):
        ks = keys_ref[:, sidx * seg:(sidx + 1) * seg]
        xs = x_ref[:, sidx * seg:(sidx + 1) * seg]
        gs.append(jnp.sum(jnp.where(ks > t, xs, 0.0), axis=1, keepdims=True))
        ls.append(jnp.sum(jnp.where(ks < u, xs, 0.0), axis=1, keepdims=True))
    s_gt = sum(gs)
    s_lt = sum(ls)
    tval = jax.lax.bitcast_convert_type(
        jnp.where(t >= 0, t, t ^ _SIGN_MASK), jnp.float32)
    uval = jax.lax.bitcast_convert_type(
        jnp.where(u >= 0, u, u ^ _SIGN_MASK), jnp.float32)
    top = s_gt + (_KMAX - cnt_gt).astype(jnp.float32) * tval
    bot = s_lt + (_KMIN - cnt_lt).astype(jnp.float32) * uval
    res = top / _KMAX + bot / _KMIN
    o_ref[0] = jnp.broadcast_to(res, (rows, 128))


from jax import lax
from jax.experimental.pallas import tpu_sc as plsc
import functools

_K = 50
_WIN = (1 << 23) - 1
_CAP = 16384
_CBUF = _CAP + 80
_NW = 32


def _splat_last(v):
    idx = jnp.full((16, 1), 15, jnp.int32)
    dn = lax.GatherDimensionNumbers(offset_dims=(), collapsed_slice_dims=(0,),
                                    start_index_map=(0,))
    return lax.gather(v, idx, dn, (1,),
                      mode=lax.GatherScatterMode.PROMISE_IN_BOUNDS)


def _splat_max(v):
    return _splat_last(plsc.cummax(v))


def _splat_min(v):
    return -_splat_last(plsc.cummax(-v))


def _splat_sum(v):
    return _splat_last(plsc.cumsum(v))


def _scal(v):
    return v[0]


def _f_to_key(v):
    b = plsc.bitcast(v, jnp.int32)
    return jnp.where(b >= 0, b, b ^ 0x7FFFFFFF)


def _key_to_f(t):
    return plsc.bitcast(jnp.where(t >= 0, t, t ^ 0x7FFFFFFF), jnp.float32)


def _count_ge(vl, nout, unroll, thf):
    zero = jnp.zeros((16,), jnp.int32)

    def body(j, acc):
        for k in range(unroll):
            acc = acc + plsc.all_reduce_population_count(
                vl(j * unroll + k) >= thf)
        return acc

    return lax.fori_loop(0, nout, body, zero)


def _select_top(vl, nout, unroll, t0, nbits):
    def sb(i, t):
        p = jnp.left_shift(jnp.int32(1), (nbits - 1) - i)
        tt = t + p
        c = _count_ge(vl, nout, unroll, _key_to_f(tt))
        ok = jnp.logical_and(tt > t, c >= _K)
        return jnp.where(ok, tt, t)

    return lax.fori_loop(0, nbits, sb, t0)


def _top_sum(vl, nout, unroll, tkey):
    thf = _key_to_f(tkey)
    zf = jnp.zeros((16,), jnp.float32)
    zi = jnp.zeros((16,), jnp.int32)

    def body(j, carry):
        s, c = carry
        for k in range(unroll):
            v = vl(j * unroll + k)
            m = v > thf
            s = s + jnp.where(m, v, 0.0)
            c = c + plsc.all_reduce_population_count(m)
        return s, c

    s, c = lax.fori_loop(0, nout, body, (zf, zi))
    return _splat_sum(s) + (_K - c).astype(jnp.float32) * thf


def _make_sc_kernel(rows, n):
    rpw = rows // _NW
    nvec = n // 16
    nout8 = nvec // 8
    mesh = plsc.VectorSubcoreMesh(core_axis_name="c", subcore_axis_name="s")

    @functools.partial(
        pl.kernel, mesh=mesh,
        compiler_params=pltpu.CompilerParams(needs_layout_passes=False),
        out_type=jax.ShapeDtypeStruct((rows * 16,), jnp.float32),
        scratch_types=[
            pltpu.VMEM((n,), jnp.float32),
            pltpu.VMEM((_CBUF,), jnp.float32),
            pltpu.VMEM((_CBUF,), jnp.float32),
            pltpu.VMEM((rpw * 16,), jnp.float32),
        ],
    )
    def sc_kernel(x_hbm, out_hbm, row_v, cand_t, cand_b, out_loc):
        wid = lax.axis_index("s") * 2 + lax.axis_index("c")
        base = wid * rpw
        iota16 = lax.iota(jnp.int32, 16)
        ninf = jnp.full((16,), -jnp.inf, jnp.float32)

        def vl_row(j):
            return row_v[pl.ds(j * 16, 16)]

        def vl_rown(j):
            return -row_v[pl.ds(j * 16, 16)]

        def vl_ct(j):
            return cand_t[pl.ds(j * 16, 16)]

        def vl_cb(j):
            return cand_b[pl.ds(j * 16, 16)]

        def row_loop(r, carry):
            pltpu.sync_copy(x_hbm.at[base + r], row_v)

            def mm(j, mc):
                mx, mn = mc
                for k in range(8):
                    v = vl_row(j * 8 + k)
                    mx = jnp.maximum(mx, v)
                    mn = jnp.minimum(mn, v)
                return mx, mn

            mx, mn = lax.fori_loop(0, nout8, mm, (ninf, -ninf))
            mxv = _splat_max(mx)
            mnv = _splat_min(mn)
            t0t = _f_to_key(mxv) - _WIN
            t0b = _f_to_key(-mnv) - _WIN
            tht = _key_to_f(t0t)
            thb = _key_to_f(t0b)

            def ap(j, bc):
                bt, bb = bc
                for k in range(8):
                    v = vl_row(j * 8 + k)
                    nv = -v
                    mt = v >= tht
                    mb = nv >= thb
                    plsc.store_compressed(
                        cand_t.at[pl.ds(jnp.minimum(bt, _CAP), 16)], v, mask=mt)
                    bt = bt + _scal(plsc.all_reduce_population_count(mt))
                    plsc.store_compressed(
                        cand_b.at[pl.ds(jnp.minimum(bb, _CAP), 16)], nv, mask=mb)
                    bb = bb + _scal(plsc.all_reduce_population_count(mb))
                return bt, bb

            zs = jnp.zeros((), jnp.int32)
            bt, bb = lax.fori_loop(0, nout8, ap, (zs, zs))
            for k in range(4):
                cand_t[pl.ds(jnp.minimum(bt + 16 * k, _CBUF - 16), 16)] = ninf
                cand_b[pl.ds(jnp.minimum(bb + 16 * k, _CBUF - 16), 16)] = ninf
            nct = bt
            ncb = bb

            def cand_sum(vl, nc, t0):
                nout4 = (nc + 63) >> 6
                tk = _select_top(vl, nout4, 4, t0, 23)
                return _top_sum(vl, nout4, 4, tk)

            def fb_sum(vl, minv):
                t0 = _f_to_key(minv)
                tk = _select_top(vl, nout8, 8, t0, 32)
                return _top_sum(vl, nout8, 8, tk)

            ts = lax.cond(
                jnp.logical_and(nct >= _K, nct <= _CAP),
                lambda _: cand_sum(vl_ct, nct, t0t),
                lambda _: fb_sum(vl_row, mnv), None)
            bs = lax.cond(
                jnp.logical_and(ncb >= _K, ncb <= _CAP),
                lambda _: cand_sum(vl_cb, ncb, t0b),
                lambda _: fb_sum(vl_rown, -mxv), None)

            res = ts / _K - bs / _K
            out_loc[pl.ds(r * 16, 16)] = res
            return carry

        lax.fori_loop(0, rpw, row_loop, 0)
        pltpu.sync_copy(out_loc, out_hbm.at[pl.ds(base * 16, rpw * 16)])

    return sc_kernel


def _tc_call(x, rows, n):
    rg = 16
    g = rows // rg
    out = pl.pallas_call(
        _select_body,
        grid=(g,),
        in_specs=[pl.BlockSpec((rg, n), lambda i: (i, 0))],
        out_specs=pl.BlockSpec((1, rg, 128), lambda i: (i, 0, 0)),
        out_shape=jax.ShapeDtypeStruct((g, rg, 128), jnp.float32),
        scratch_shapes=[pltpu.VMEM((rg, n), jnp.int32)],
    )(x)
    return out[:, :, 0].reshape(rows)


def kernel(inputs):
    b, h, w, c = inputs.shape
    n = h * w
    ct = 52
    rt = b * ct
    rs = b * (c - ct)
    x_sc = jnp.transpose(inputs[:, :, :, ct:], (0, 3, 1, 2)).reshape(rs, n)
    x_tc = jnp.transpose(inputs[:, :, :, :ct], (0, 3, 1, 2)).reshape(rt, n)
    out_sc = _make_sc_kernel(rs, n)(x_sc)
    out_tc = _tc_call(x_tc, rt, n)
    out = jnp.concatenate([out_tc.reshape(b, ct),
                           out_sc.reshape(rs, 16)[:, 0].reshape(b, c - ct)],
                          axis=1)
    return out
